# baseline (speedup 1.0000x reference)
import functools

import jax
import jax.numpy as jnp
from jax import lax
from jax.experimental import pallas as pl
from jax.experimental.pallas import tpu as pltpu
from jax.experimental.pallas import tpu_sc as plsc

N = 10000
E = 320000
B = 16
VD = 128
ED = 16
HD = 128
KH = 4

NC = 2
NS = 16
NW = NC * NS
EPW = E // NW
CH = 64
NCH = EPW // CH
TAIL = EPW - NCH * CH
UN = 12
NP = 10240
RPS = NP // NS

EBLK = 4000
NB = 2000


def _ek_body(e_ref, kw_ref, kb_ref, out_ref):
    out_ref[...] = (
        jnp.dot(e_ref[...], kw_ref[...], preferred_element_type=jnp.float32)
        + kb_ref[...]
    )


def _compute_ek(e, kw, kb):
    return pl.pallas_call(
        _ek_body,
        grid=(E // EBLK,),
        in_specs=[
            pl.BlockSpec((EBLK, ED), lambda i: (i, 0)),
            pl.BlockSpec((ED, HD), lambda i: (0, 0)),
            pl.BlockSpec((1, HD), lambda i: (0, 0)),
        ],
        out_specs=pl.BlockSpec((EBLK, HD), lambda i: (i, 0)),
        out_shape=jax.ShapeDtypeStruct((E, HD), jnp.float32),
    )(e, kw, kb.reshape(1, HD))


def _sc_edge_body(v_hbm, ek_hbm, src_hbm, dst_hbm, zero_hbm, out_hbm,
                  idx0, idx1, idx2, idx3, ekv0, ekv1, ekv2,
                  vsrc0, vsrc1, idxT, acc,
                  semI0, semI1, semI2, semI3, semE0, semE1, semE2,
                  semG0, semG1, semS0, semS1, semS2):
    cid = lax.axis_index("c")
    sid = lax.axis_index("s")
    wid = sid * NC + cid
    wbase = wid * EPW

    idx = (idx0, idx1, idx2, idx3)
    ekv = (ekv0, ekv1, ekv2)
    vsrc = (vsrc0, vsrc1)
    semI = (semI0, semI1, semI2, semI3)
    semE = (semE0, semE1, semE2)
    semG = (semG0, semG1)
    semS = (semS0, semS1, semS2)

    pltpu.sync_copy(zero_hbm.at[pl.ds(sid * RPS, RPS)],
                    acc.at[pl.ds(sid * RPS, RPS)])
    plsc.subcore_barrier()

    def issue_i(ci, p):
        base = pl.multiple_of(wbase + ci * CH, 8)
        pltpu.async_copy(src_hbm.at[pl.ds(base, CH)], idx[p].at[0], semI[p])
        pltpu.async_copy(dst_hbm.at[pl.ds(base, CH)], idx[p].at[1], semI[p])

    def wait_i(p):
        pltpu.make_async_copy(src_hbm.at[pl.ds(0, CH)], idx[p].at[0],
                              semI[p]).wait()
        pltpu.make_async_copy(dst_hbm.at[pl.ds(0, CH)], idx[p].at[1],
                              semI[p]).wait()

    def issue_e(ci, p):
        base = pl.multiple_of(wbase + ci * CH, 8)
        pltpu.async_copy(ek_hbm.at[pl.ds(base, CH)], ekv[p], semE[p])

    def wait_e(p):
        pltpu.make_async_copy(ek_hbm.at[pl.ds(0, CH)], ekv[p],
                              semE[p]).wait()

    def issue_g(pi, pg):
        pltpu.async_copy(v_hbm.at[idx[pi].at[0]], vsrc[pg], semG[pg])

    def wait_g(pi, pg):
        pltpu.make_async_copy(v_hbm.at[idx[pi].at[0]], vsrc[pg],
                              semG[pg]).wait()

    def drain_s(pe, pi):
        pltpu.make_async_copy(ekv[pe], acc.at[idx[pi].at[1]],
                              semS[pe]).wait()

    def compute(ebuf, sbuf, n):
        def edge_body(j, c2):
            for k in range(HD // 16):
                sl = pl.ds(k * 16, 16)
                x = ebuf[j, sl] * sbuf[j, sl]
                ebuf[j, sl] = jnp.maximum(x, x * 0.1)
            return c2
        lax.fori_loop(0, n, edge_body, 0, unroll=2)

    def step(ci, pI, pE, pG, drain_pred, idx2_ok, nxt_ok):
        def do_drain():
            drain_s((pE + 1) % 3, (pI + 2) % 4)

        if drain_pred is None:
            do_drain()
        else:
            @pl.when(drain_pred)
            def _():
                do_drain()

        def do_idx2():
            issue_i(ci + 2, (pI + 2) % 4)

        if idx2_ok is None:
            do_idx2()
        else:
            @pl.when(idx2_ok)
            def _():
                do_idx2()

        def do_nxt():
            issue_e(ci + 1, (pE + 1) % 3)
            wait_i((pI + 1) % 4)
            issue_g((pI + 1) % 4, 1 - pG)

        if nxt_ok is None:
            do_nxt()
        else:
            @pl.when(nxt_ok)
            def _():
                do_nxt()

        wait_g(pI, pG)
        wait_e(pE)
        compute(ekv[pE], vsrc[pG], CH)
        pltpu.async_copy(ekv[pE], acc.at[idx[pI].at[1]], semS[pE], add=True)

    issue_i(0, 0)
    issue_i(1, 1)
    issue_e(0, 0)
    wait_i(0)
    issue_g(0, 0)

    def twelve_body(k, carry):
        for j in range(UN):
            pred_drain = (k >= 1) if j < 2 else None
            pred_idx2 = (k < NCH // UN - 1) if j >= UN - 2 else None
            pred_nxt = (k < NCH // UN - 1) if j == UN - 1 else None
            step(UN * k + j, j % 4, j % 3, j % 2,
                 pred_drain, pred_idx2, pred_nxt)
        return carry

    lax.fori_loop(0, NCH // UN, twelve_body, 0, unroll=False)
    drain_s((NCH - 2) % 3, (NCH - 2) % 4)
    drain_s((NCH - 1) % 3, (NCH - 1) % 4)

    tbase = pl.multiple_of(wbase + NCH * CH, 8)
    pltpu.sync_copy(src_hbm.at[pl.ds(tbase, TAIL)], idxT.at[0])
    pltpu.sync_copy(dst_hbm.at[pl.ds(tbase, TAIL)], idxT.at[1])
    pltpu.sync_copy(ek_hbm.at[pl.ds(tbase, TAIL)], ekv0.at[pl.ds(0, TAIL)])
    pltpu.async_copy(v_hbm.at[idxT.at[0]], vsrc0.at[pl.ds(0, TAIL)],
                     semG0).wait()
    compute(ekv0, vsrc0, TAIL)
    pltpu.sync_copy(ekv0.at[pl.ds(0, TAIL)], acc.at[idxT.at[1]], add=True)

    plsc.subcore_barrier()
    pltpu.sync_copy(acc.at[pl.ds(sid * RPS, RPS)],
                    out_hbm.at[pl.ds(cid * NP + sid * RPS, RPS)])


def _sc_edge_sum(v, ek, src, dst, zero):
    mesh = plsc.VectorSubcoreMesh(core_axis_name="c", subcore_axis_name="s",
                                  num_cores=NC, num_subcores=NS)
    fn = pl.kernel(
        _sc_edge_body,
        out_type=jax.ShapeDtypeStruct((NC * NP, HD), jnp.float32),
        mesh=mesh,
        scratch_types=(
            [pltpu.VMEM((2, CH), jnp.int32)] * 4
            + [pltpu.VMEM((CH, HD), jnp.float32)] * 3
            + [pltpu.VMEM((CH, HD), jnp.float32)] * 2
            + [
                pltpu.VMEM((2, TAIL), jnp.int32),
                pltpu.VMEM_SHARED((NP, HD), jnp.float32),
            ]
            + [pltpu.SemaphoreType.DMA] * 12
        ),
    )
    return fn(v, ek, src, dst, zero)


def _heads_body(v_ref, s_ref, gid_ref, haw, hab, hbw, hbb, hcw, hcb,
                hdw, hdb, out_ref):
    v = v_ref[...]
    s = s_ref[...]
    gid = gid_ref[...]
    oh = (gid == lax.broadcasted_iota(jnp.int32, (1, B), 1)).astype(
        jnp.float32)
    dn = jnp.tanh(jnp.dot(v, haw[0], preferred_element_type=jnp.float32)
                  + hab[0])
    ds16 = jnp.tanh(jnp.dot(s, hbw[0], preferred_element_type=jnp.float32)
                    + hbb[0])
    dsup = jnp.dot(oh, ds16, preferred_element_type=jnp.float32)
    a = jnp.dot(dn * dsup, hcw[0], preferred_element_type=jnp.float32) \
        + hcb[0]
    am = jnp.where(oh > 0.0, a, -3e38)
    mx = jnp.max(am, axis=0, keepdims=True)
    mxn = jnp.sum(oh * mx, axis=1, keepdims=True)
    ex = jnp.exp(a - mxn)
    sm = jnp.sum(jnp.where(oh > 0.0, ex, 0.0), axis=0, keepdims=True)
    smn = jnp.sum(oh * sm, axis=1, keepdims=True)
    attn = ex / smn
    h = (jnp.dot(v, hdw[0], preferred_element_type=jnp.float32)
         + hdb[0]) * attn
    out_ref[0] = lax.dot_general(
        oh, h, dimension_numbers=(((0,), (0,)), ((), ())),
        preferred_element_type=jnp.float32)


def _compute_heads(v, s, gid2d, p):
    return pl.pallas_call(
        _heads_body,
        grid=(KH,),
        in_specs=[
            pl.BlockSpec((N, VD), lambda i: (0, 0)),
            pl.BlockSpec((B, VD), lambda i: (0, 0)),
            pl.BlockSpec((N, 1), lambda i: (0, 0)),
            pl.BlockSpec((1, VD, HD), lambda i: (i, 0, 0)),
            pl.BlockSpec((1, 1, HD), lambda i: (i, 0, 0)),
            pl.BlockSpec((1, VD, HD), lambda i: (i, 0, 0)),
            pl.BlockSpec((1, 1, HD), lambda i: (i, 0, 0)),
            pl.BlockSpec((1, HD, 1), lambda i: (i, 0, 0)),
            pl.BlockSpec((1, 1, 1), lambda i: (i, 0, 0)),
            pl.BlockSpec((1, VD, HD), lambda i: (i, 0, 0)),
            pl.BlockSpec((1, 1, HD), lambda i: (i, 0, 0)),
        ],
        out_specs=pl.BlockSpec((1, B, HD), lambda i: (i, 0, 0)),
        out_shape=jax.ShapeDtypeStruct((KH, B, HD), jnp.float32),
    )(v, s, gid2d,
      p['hA_w'], p['hA_b'].reshape(KH, 1, HD),
      p['hB_w'], p['hB_b'].reshape(KH, 1, HD),
      p['hC_w'], p['hC_b'].reshape(KH, 1, 1),
      p['hD_w'], p['hD_b'].reshape(KH, 1, HD))


def _gru(x, h, wih, bih, whh, bhh):
    gi = jnp.dot(x, wih, preferred_element_type=jnp.float32) + bih
    gh = jnp.dot(h, whh, preferred_element_type=jnp.float32) + bhh
    r = jax.nn.sigmoid(gi[:, 0:HD] + gh[:, 0:HD])
    z = jax.nn.sigmoid(gi[:, HD:2 * HD] + gh[:, HD:2 * HD])
    n = jnp.tanh(gi[:, 2 * HD:] + r * gh[:, 2 * HD:])
    return (1.0 - z) * n + z * h


def _gate(a_, b_, c_, aw, ab, bw, bb, wih, bih, whh, bhh):
    z = jax.nn.sigmoid(
        jnp.dot(a_, aw, preferred_element_type=jnp.float32) + ab
        + jnp.dot(b_, bw, preferred_element_type=jnp.float32) + bb)
    hh = z * b_ + (1.0 - z) * a_
    return _gru(c_, hh, wih, bih, whh, bhh)


def _final_body(v_ref, svp_ref, gid_ref, s_ref, heads_ref,
                ew, eb, cw, cb, aw, ab, bw, bb,
                gmaw, gmab, gmbw, gmbb, gmwih, gmbih, gmwhh, gmbhh,
                gsaw, gsab, gsbw, gsbb, gswih, gsbih, gswhh, gsbhh,
                vv_ref, ss_ref):
    v = v_ref[...]
    gid = gid_ref[...]
    s = s_ref[...]
    oh = (gid == lax.broadcasted_iota(jnp.int32, (1, B), 1)).astype(
        jnp.float32)
    s2m16 = jnp.tanh(jnp.dot(s, cw[...], preferred_element_type=jnp.float32)
                     + cb[...])
    s2m = jnp.dot(oh, s2m16, preferred_element_type=jnp.float32)
    sve = svp_ref[0] + svp_ref[1]
    m2m = (jnp.dot(sve, ew[0:HD], preferred_element_type=jnp.float32)
           + jnp.dot(v, ew[HD:], preferred_element_type=jnp.float32)
           + eb[...])
    m2m = jnp.maximum(m2m, m2m * 0.1)
    vv_ref[...] = _gate(m2m, s2m, v,
                        gmaw[...], gmab[...], gmbw[...], gmbb[...],
                        gmwih[...], gmbih[...], gmwhh[...], gmbhh[...])

    @pl.when(pl.program_id(0) == 0)
    def _():
        hcat = jnp.concatenate(
            [heads_ref[i] for i in range(KH)], axis=1)
        m2s = jnp.tanh(jnp.dot(hcat, bw[...],
                               preferred_element_type=jnp.float32) + bb[...])
        s2s = jnp.tanh(jnp.dot(s, aw[...],
                               preferred_element_type=jnp.float32) + ab[...])
        ss_ref[...] = _gate(s2s, m2s, s,
                            gsaw[...], gsab[...], gsbw[...], gsbb[...],
                            gswih[...], gsbih[...], gswhh[...], gsbhh[...])


def _compute_final(v, svp, gid2d, s, heads, p):
    fixed = lambda *shape: pl.BlockSpec(shape, lambda i: (0,) * len(shape))
    return pl.pallas_call(
        _final_body,
        grid=(N // NB,),
        in_specs=[
            pl.BlockSpec((NB, VD), lambda i: (i, 0)),
            pl.BlockSpec((2, NB, HD), lambda i: (0, i, 0)),
            pl.BlockSpec((NB, 1), lambda i: (i, 0)),
            fixed(B, VD),
            fixed(KH, B, HD),
            fixed(HD + VD, HD),
            fixed(1, HD),
            fixed(VD, HD),
            fixed(1, HD),
            fixed(VD, HD),
            fixed(1, HD),
            fixed(KH * HD, HD),
            fixed(1, HD),
            fixed(HD, HD), fixed(1, HD), fixed(HD, HD), fixed(1, HD),
            fixed(HD, 3 * HD), fixed(1, 3 * HD),
            fixed(HD, 3 * HD), fixed(1, 3 * HD),
            fixed(HD, HD), fixed(1, HD), fixed(HD, HD), fixed(1, HD),
            fixed(HD, 3 * HD), fixed(1, 3 * HD),
            fixed(HD, 3 * HD), fixed(1, 3 * HD),
        ],
        out_specs=[
            pl.BlockSpec((NB, HD), lambda i: (i, 0)),
            pl.BlockSpec((B, HD), lambda i: (0, 0)),
        ],
        out_shape=[
            jax.ShapeDtypeStruct((N, HD), jnp.float32),
            jax.ShapeDtypeStruct((B, HD), jnp.float32),
        ],
    )(v, svp, gid2d, s, heads,
      p['E_w'], p['E_b'].reshape(1, HD),
      p['C_w'], p['C_b'].reshape(1, HD),
      p['A_w'], p['A_b'].reshape(1, HD),
      p['B_w'], p['B_b'].reshape(1, HD),
      p['gm_A_w'], p['gm_A_b'].reshape(1, HD),
      p['gm_B_w'], p['gm_B_b'].reshape(1, HD),
      p['gm_Wih'], p['gm_bih'].reshape(1, 3 * HD),
      p['gm_Whh'], p['gm_bhh'].reshape(1, 3 * HD),
      p['gs_A_w'], p['gs_A_b'].reshape(1, HD),
      p['gs_B_w'], p['gs_B_b'].reshape(1, HD),
      p['gs_Wih'], p['gs_bih'].reshape(1, 3 * HD),
      p['gs_Whh'], p['gs_bhh'].reshape(1, 3 * HD))


def kernel(v, edge_index, e, s, node_graph_ids, params):
    v = v.astype(jnp.float32)
    e = e.astype(jnp.float32)
    s = s.astype(jnp.float32)
    src = edge_index[0].astype(jnp.int32)
    dst = edge_index[1].astype(jnp.int32)
    gid2d = node_graph_ids.astype(jnp.int32).reshape(N, 1)
    zero = jnp.zeros((NP, HD), jnp.float32)

    ek = _compute_ek(e, params['K_w'], params['K_b'])
    svp_flat = _sc_edge_sum(v, ek, src, dst, zero)
    svp = svp_flat.reshape(NC, NP, HD)[:, :N, :]
    heads = _compute_heads(v, s, gid2d, params)
    vv, ss = _compute_final(v, svp, gid2d, s, heads, params)
    return (vv, ss)


# speedup vs baseline: 1.7353x; 1.7353x over previous
import functools

import jax
import jax.numpy as jnp
from jax import lax
from jax.experimental import pallas as pl
from jax.experimental.pallas import tpu as pltpu
from jax.experimental.pallas import tpu_sc as plsc

N = 10000
E = 320000
B = 16
VD = 128
ED = 16
HD = 128
KH = 4

NC = 2
NS = 16
NW = NC * NS
EPW = E // NW
CH = 64
NCH = EPW // CH
TAIL = EPW - NCH * CH
UN = 12
NP = 10240
RPS = NP // NS

EBLK = 4000
NB = 2000


def _ek_body(e_ref, kw_ref, kb_ref, out_ref):
    out_ref[...] = (
        jnp.dot(e_ref[...], kw_ref[...], preferred_element_type=jnp.float32)
        + kb_ref[...]
    )


def _compute_ek(e, kw, kb):
    return pl.pallas_call(
        _ek_body,
        grid=(E // EBLK,),
        in_specs=[
            pl.BlockSpec((EBLK, ED), lambda i: (i, 0)),
            pl.BlockSpec((ED, HD), lambda i: (0, 0)),
            pl.BlockSpec((1, HD), lambda i: (0, 0)),
        ],
        out_specs=pl.BlockSpec((EBLK, HD), lambda i: (i, 0)),
        out_shape=jax.ShapeDtypeStruct((E, HD), jnp.float32),
    )(e, kw, kb.reshape(1, HD))


def _sc_edge_body(v_hbm, ek_hbm, src_hbm, dst_hbm, zero_hbm, out_hbm,
                  idx0, idx1, idx2, idx3, ekv0, ekv1, ekv2,
                  vsrc0, vsrc1, idxT, acc,
                  semI0, semI1, semI2, semI3, semE0, semE1, semE2,
                  semG0, semG1, semS0, semS1, semS2):
    cid = lax.axis_index("c")
    sid = lax.axis_index("s")
    wid = sid * NC + cid
    wbase = wid * EPW

    idx = (idx0, idx1, idx2, idx3)
    ekv = (ekv0, ekv1, ekv2)
    vsrc = (vsrc0, vsrc1)
    semI = (semI0, semI1, semI2, semI3)
    semE = (semE0, semE1, semE2)
    semG = (semG0, semG1)
    semS = (semS0, semS1, semS2)

    pltpu.sync_copy(zero_hbm.at[pl.ds(sid * RPS, RPS)],
                    acc.at[pl.ds(sid * RPS, RPS)])
    plsc.subcore_barrier()

    def issue_i(ci, p):
        base = pl.multiple_of(wbase + ci * CH, 8)
        pltpu.async_copy(src_hbm.at[pl.ds(base, CH)], idx[p].at[0], semI[p])
        pltpu.async_copy(dst_hbm.at[pl.ds(base, CH)], idx[p].at[1], semI[p])

    def wait_i(p):
        pltpu.make_async_copy(src_hbm.at[pl.ds(0, CH)], idx[p].at[0],
                              semI[p]).wait()
        pltpu.make_async_copy(dst_hbm.at[pl.ds(0, CH)], idx[p].at[1],
                              semI[p]).wait()

    def issue_e(ci, p):
        base = pl.multiple_of(wbase + ci * CH, 8)
        pltpu.async_copy(ek_hbm.at[pl.ds(base, CH)], ekv[p], semE[p])

    def wait_e(p):
        pltpu.make_async_copy(ek_hbm.at[pl.ds(0, CH)], ekv[p],
                              semE[p]).wait()

    def issue_g(pi, pg):
        pltpu.async_copy(v_hbm.at[idx[pi].at[0]], vsrc[pg], semG[pg])

    def wait_g(pi, pg):
        pltpu.make_async_copy(v_hbm.at[idx[pi].at[0]], vsrc[pg],
                              semG[pg]).wait()

    def drain_s(pe, pi):
        pltpu.make_async_copy(ekv[pe], acc.at[idx[pi].at[1]],
                              semS[pe]).wait()

    def compute(ebuf, sbuf, n):
        @plsc.parallel_loop(0, n, 1)
        def edge_body(j):
            for k in range(HD // 16):
                sl = pl.ds(k * 16, 16)
                x = ebuf[j, sl] * sbuf[j, sl]
                ebuf[j, sl] = jnp.maximum(x, x * 0.1)

    def step(ci, pI, pE, pG, drain_pred, idx2_ok, nxt_ok):
        def do_drain():
            drain_s((pE + 1) % 3, (pI + 2) % 4)

        if drain_pred is None:
            do_drain()
        else:
            @pl.when(drain_pred)
            def _():
                do_drain()

        def do_idx2():
            issue_i(ci + 2, (pI + 2) % 4)

        if idx2_ok is None:
            do_idx2()
        else:
            @pl.when(idx2_ok)
            def _():
                do_idx2()

        def do_nxt():
            issue_e(ci + 1, (pE + 1) % 3)
            wait_i((pI + 1) % 4)
            issue_g((pI + 1) % 4, 1 - pG)

        if nxt_ok is None:
            do_nxt()
        else:
            @pl.when(nxt_ok)
            def _():
                do_nxt()

        wait_g(pI, pG)
        wait_e(pE)
        compute(ekv[pE], vsrc[pG], CH)
        pltpu.async_copy(ekv[pE], acc.at[idx[pI].at[1]], semS[pE], add=True)

    issue_i(0, 0)
    issue_i(1, 1)
    issue_e(0, 0)
    wait_i(0)
    issue_g(0, 0)

    def twelve_body(k, carry):
        for j in range(UN):
            pred_drain = (k >= 1) if j < 2 else None
            pred_idx2 = (k < NCH // UN - 1) if j >= UN - 2 else None
            pred_nxt = (k < NCH // UN - 1) if j == UN - 1 else None
            step(UN * k + j, j % 4, j % 3, j % 2,
                 pred_drain, pred_idx2, pred_nxt)
        return carry

    lax.fori_loop(0, NCH // UN, twelve_body, 0, unroll=False)
    drain_s((NCH - 2) % 3, (NCH - 2) % 4)
    drain_s((NCH - 1) % 3, (NCH - 1) % 4)

    tbase = pl.multiple_of(wbase + NCH * CH, 8)
    pltpu.sync_copy(src_hbm.at[pl.ds(tbase, TAIL)], idxT.at[0])
    pltpu.sync_copy(dst_hbm.at[pl.ds(tbase, TAIL)], idxT.at[1])
    pltpu.sync_copy(ek_hbm.at[pl.ds(tbase, TAIL)], ekv0.at[pl.ds(0, TAIL)])
    pltpu.async_copy(v_hbm.at[idxT.at[0]], vsrc0.at[pl.ds(0, TAIL)],
                     semG0).wait()
    compute(ekv0, vsrc0, TAIL)
    pltpu.sync_copy(ekv0.at[pl.ds(0, TAIL)], acc.at[idxT.at[1]], add=True)

    plsc.subcore_barrier()
    pltpu.sync_copy(acc.at[pl.ds(sid * RPS, RPS)],
                    out_hbm.at[pl.ds(cid * NP + sid * RPS, RPS)])


def _sc_edge_sum(v, ek, src, dst, zero):
    mesh = plsc.VectorSubcoreMesh(core_axis_name="c", subcore_axis_name="s",
                                  num_cores=NC, num_subcores=NS)
    fn = pl.kernel(
        _sc_edge_body,
        out_type=jax.ShapeDtypeStruct((NC * NP, HD), jnp.float32),
        mesh=mesh,
        scratch_types=(
            [pltpu.VMEM((2, CH), jnp.int32)] * 4
            + [pltpu.VMEM((CH, HD), jnp.float32)] * 3
            + [pltpu.VMEM((CH, HD), jnp.float32)] * 2
            + [
                pltpu.VMEM((2, TAIL), jnp.int32),
                pltpu.VMEM_SHARED((NP, HD), jnp.float32),
            ]
            + [pltpu.SemaphoreType.DMA] * 12
        ),
    )
    return fn(v, ek, src, dst, zero)


def _heads_body(v_ref, s_ref, gid_ref, haw, hab, hbw, hbb, hcw, hcb,
                hdw, hdb, out_ref):
    v = v_ref[...]
    s = s_ref[...]
    gid = gid_ref[...]
    oh = (gid == lax.broadcasted_iota(jnp.int32, (1, B), 1)).astype(
        jnp.float32)
    dn = jnp.tanh(jnp.dot(v, haw[0], preferred_element_type=jnp.float32)
                  + hab[0])
    ds16 = jnp.tanh(jnp.dot(s, hbw[0], preferred_element_type=jnp.float32)
                    + hbb[0])
    dsup = jnp.dot(oh, ds16, preferred_element_type=jnp.float32)
    a = jnp.dot(dn * dsup, hcw[0], preferred_element_type=jnp.float32) \
        + hcb[0]
    am = jnp.where(oh > 0.0, a, -3e38)
    mx = jnp.max(am, axis=0, keepdims=True)
    mxn = jnp.sum(oh * mx, axis=1, keepdims=True)
    ex = jnp.exp(a - mxn)
    sm = jnp.sum(jnp.where(oh > 0.0, ex, 0.0), axis=0, keepdims=True)
    smn = jnp.sum(oh * sm, axis=1, keepdims=True)
    attn = ex / smn
    h = (jnp.dot(v, hdw[0], preferred_element_type=jnp.float32)
         + hdb[0]) * attn
    out_ref[0] = lax.dot_general(
        oh, h, dimension_numbers=(((0,), (0,)), ((), ())),
        preferred_element_type=jnp.float32)


def _compute_heads(v, s, gid2d, p):
    return pl.pallas_call(
        _heads_body,
        grid=(KH,),
        in_specs=[
            pl.BlockSpec((N, VD), lambda i: (0, 0)),
            pl.BlockSpec((B, VD), lambda i: (0, 0)),
            pl.BlockSpec((N, 1), lambda i: (0, 0)),
            pl.BlockSpec((1, VD, HD), lambda i: (i, 0, 0)),
            pl.BlockSpec((1, 1, HD), lambda i: (i, 0, 0)),
            pl.BlockSpec((1, VD, HD), lambda i: (i, 0, 0)),
            pl.BlockSpec((1, 1, HD), lambda i: (i, 0, 0)),
            pl.BlockSpec((1, HD, 1), lambda i: (i, 0, 0)),
            pl.BlockSpec((1, 1, 1), lambda i: (i, 0, 0)),
            pl.BlockSpec((1, VD, HD), lambda i: (i, 0, 0)),
            pl.BlockSpec((1, 1, HD), lambda i: (i, 0, 0)),
        ],
        out_specs=pl.BlockSpec((1, B, HD), lambda i: (i, 0, 0)),
        out_shape=jax.ShapeDtypeStruct((KH, B, HD), jnp.float32),
    )(v, s, gid2d,
      p['hA_w'], p['hA_b'].reshape(KH, 1, HD),
      p['hB_w'], p['hB_b'].reshape(KH, 1, HD),
      p['hC_w'], p['hC_b'].reshape(KH, 1, 1),
      p['hD_w'], p['hD_b'].reshape(KH, 1, HD))


def _gru(x, h, wih, bih, whh, bhh):
    gi = jnp.dot(x, wih, preferred_element_type=jnp.float32) + bih
    gh = jnp.dot(h, whh, preferred_element_type=jnp.float32) + bhh
    r = jax.nn.sigmoid(gi[:, 0:HD] + gh[:, 0:HD])
    z = jax.nn.sigmoid(gi[:, HD:2 * HD] + gh[:, HD:2 * HD])
    n = jnp.tanh(gi[:, 2 * HD:] + r * gh[:, 2 * HD:])
    return (1.0 - z) * n + z * h


def _gate(a_, b_, c_, aw, ab, bw, bb, wih, bih, whh, bhh):
    z = jax.nn.sigmoid(
        jnp.dot(a_, aw, preferred_element_type=jnp.float32) + ab
        + jnp.dot(b_, bw, preferred_element_type=jnp.float32) + bb)
    hh = z * b_ + (1.0 - z) * a_
    return _gru(c_, hh, wih, bih, whh, bhh)


def _final_body(v_ref, svp_ref, gid_ref, s_ref, heads_ref,
                ew, eb, cw, cb, aw, ab, bw, bb,
                gmaw, gmab, gmbw, gmbb, gmwih, gmbih, gmwhh, gmbhh,
                gsaw, gsab, gsbw, gsbb, gswih, gsbih, gswhh, gsbhh,
                vv_ref, ss_ref):
    v = v_ref[...]
    gid = gid_ref[...]
    s = s_ref[...]
    oh = (gid == lax.broadcasted_iota(jnp.int32, (1, B), 1)).astype(
        jnp.float32)
    s2m16 = jnp.tanh(jnp.dot(s, cw[...], preferred_element_type=jnp.float32)
                     + cb[...])
    s2m = jnp.dot(oh, s2m16, preferred_element_type=jnp.float32)
    sve = svp_ref[0] + svp_ref[1]
    m2m = (jnp.dot(sve, ew[0:HD], preferred_element_type=jnp.float32)
           + jnp.dot(v, ew[HD:], preferred_element_type=jnp.float32)
           + eb[...])
    m2m = jnp.maximum(m2m, m2m * 0.1)
    vv_ref[...] = _gate(m2m, s2m, v,
                        gmaw[...], gmab[...], gmbw[...], gmbb[...],
                        gmwih[...], gmbih[...], gmwhh[...], gmbhh[...])

    @pl.when(pl.program_id(0) == 0)
    def _():
        hcat = jnp.concatenate(
            [heads_ref[i] for i in range(KH)], axis=1)
        m2s = jnp.tanh(jnp.dot(hcat, bw[...],
                               preferred_element_type=jnp.float32) + bb[...])
        s2s = jnp.tanh(jnp.dot(s, aw[...],
                               preferred_element_type=jnp.float32) + ab[...])
        ss_ref[...] = _gate(s2s, m2s, s,
                            gsaw[...], gsab[...], gsbw[...], gsbb[...],
                            gswih[...], gsbih[...], gswhh[...], gsbhh[...])


def _compute_final(v, svp, gid2d, s, heads, p):
    fixed = lambda *shape: pl.BlockSpec(shape, lambda i: (0,) * len(shape))
    return pl.pallas_call(
        _final_body,
        grid=(N // NB,),
        in_specs=[
            pl.BlockSpec((NB, VD), lambda i: (i, 0)),
            pl.BlockSpec((2, NB, HD), lambda i: (0, i, 0)),
            pl.BlockSpec((NB, 1), lambda i: (i, 0)),
            fixed(B, VD),
            fixed(KH, B, HD),
            fixed(HD + VD, HD),
            fixed(1, HD),
            fixed(VD, HD),
            fixed(1, HD),
            fixed(VD, HD),
            fixed(1, HD),
            fixed(KH * HD, HD),
            fixed(1, HD),
            fixed(HD, HD), fixed(1, HD), fixed(HD, HD), fixed(1, HD),
            fixed(HD, 3 * HD), fixed(1, 3 * HD),
            fixed(HD, 3 * HD), fixed(1, 3 * HD),
            fixed(HD, HD), fixed(1, HD), fixed(HD, HD), fixed(1, HD),
            fixed(HD, 3 * HD), fixed(1, 3 * HD),
            fixed(HD, 3 * HD), fixed(1, 3 * HD),
        ],
        out_specs=[
            pl.BlockSpec((NB, HD), lambda i: (i, 0)),
            pl.BlockSpec((B, HD), lambda i: (0, 0)),
        ],
        out_shape=[
            jax.ShapeDtypeStruct((N, HD), jnp.float32),
            jax.ShapeDtypeStruct((B, HD), jnp.float32),
        ],
    )(v, svp, gid2d, s, heads,
      p['E_w'], p['E_b'].reshape(1, HD),
      p['C_w'], p['C_b'].reshape(1, HD),
      p['A_w'], p['A_b'].reshape(1, HD),
      p['B_w'], p['B_b'].reshape(1, HD),
      p['gm_A_w'], p['gm_A_b'].reshape(1, HD),
      p['gm_B_w'], p['gm_B_b'].reshape(1, HD),
      p['gm_Wih'], p['gm_bih'].reshape(1, 3 * HD),
      p['gm_Whh'], p['gm_bhh'].reshape(1, 3 * HD),
      p['gs_A_w'], p['gs_A_b'].reshape(1, HD),
      p['gs_B_w'], p['gs_B_b'].reshape(1, HD),
      p['gs_Wih'], p['gs_bih'].reshape(1, 3 * HD),
      p['gs_Whh'], p['gs_bhh'].reshape(1, 3 * HD))


def kernel(v, edge_index, e, s, node_graph_ids, params):
    v = v.astype(jnp.float32)
    e = e.astype(jnp.float32)
    s = s.astype(jnp.float32)
    src = edge_index[0].astype(jnp.int32)
    dst = edge_index[1].astype(jnp.int32)
    gid2d = node_graph_ids.astype(jnp.int32).reshape(N, 1)
    zero = jnp.zeros((NP, HD), jnp.float32)

    ek = _compute_ek(e, params['K_w'], params['K_b'])
    svp_flat = _sc_edge_sum(v, ek, src, dst, zero)
    svp = svp_flat.reshape(NC, NP, HD)[:, :N, :]
    heads = _compute_heads(v, s, gid2d, params)
    vv, ss = _compute_final(v, svp, gid2d, s, heads, params)
    return (vv, ss)
